# baseline (speedup 1.0000x reference)
import functools

import jax
import jax.numpy as jnp
from jax import lax
from jax.experimental import pallas as pl
from jax.experimental.pallas import tpu as pltpu
from jax.experimental.pallas import tpu_sc as plsc

B = 16384
EMB = 32
HID = 32
NC = 2
NS = 16
NW = NC * NS
BPW = B // NW
CHUNK = 128
NCHUNK = BPW // CHUNK


def _sc_gather(u_idx2d, m_idx2d, user_table, movie_table):
    mesh = plsc.VectorSubcoreMesh(core_axis_name="c", subcore_axis_name="s")

    @functools.partial(
        pl.kernel,
        mesh=mesh,
        compiler_params=pltpu.CompilerParams(use_tc_tiling_on_sc=False),
        out_type=(
            jax.ShapeDtypeStruct((B, EMB), jnp.float32),
            jax.ShapeDtypeStruct((B, EMB), jnp.float32),
        ),
        scratch_types=[
            pltpu.VMEM((NCHUNK, CHUNK), jnp.int32),
            pltpu.VMEM((NCHUNK, CHUNK), jnp.int32),
            pltpu.VMEM((BPW, EMB), jnp.float32),
            pltpu.VMEM((BPW, EMB), jnp.float32),
            pltpu.SemaphoreType.DMA,
            pltpu.SemaphoreType.DMA,
        ],
    )
    def k(u_idx_hbm, m_idx_hbm, ut_hbm, mt_hbm, u_out, m_out,
          uidx_v, midx_v, urows_v, mrows_v, sem_u, sem_m):
        wid = lax.axis_index("s") * NC + lax.axis_index("c")
        base = wid * BPW
        pltpu.sync_copy(u_idx_hbm.at[pl.ds(wid * NCHUNK, NCHUNK)], uidx_v)
        pltpu.sync_copy(m_idx_hbm.at[pl.ds(wid * NCHUNK, NCHUNK)], midx_v)
        copies = []
        for j in range(NCHUNK):
            copies.append(pltpu.async_copy(
                ut_hbm.at[uidx_v.at[j]],
                urows_v.at[pl.ds(j * CHUNK, CHUNK)], sem_u))
            copies.append(pltpu.async_copy(
                mt_hbm.at[midx_v.at[j]],
                mrows_v.at[pl.ds(j * CHUNK, CHUNK)], sem_m))
        for c in copies:
            c.wait()
        pltpu.sync_copy(urows_v, u_out.at[pl.ds(base, BPW)])
        pltpu.sync_copy(mrows_v, m_out.at[pl.ds(base, BPW)])

    return k(u_idx2d, m_idx2d, user_table, movie_table)


def _tc_mlp(u_rows, m_rows, w1u_t, w1m_t, b1_2d, w2_2d, b2_2d):
    BLK = 2048

    def body(u_ref, m_ref, w1u_ref, w1m_ref, b1_ref, w2_ref, b2_ref, o_ref):
        h = jnp.dot(u_ref[...], w1u_ref[...], preferred_element_type=jnp.float32)
        h = h + jnp.dot(m_ref[...], w1m_ref[...], preferred_element_type=jnp.float32)
        h = jnp.maximum(h + b1_ref[...], 0.0)
        o = jnp.sum(h * w2_ref[...], axis=1, keepdims=True) + b2_ref[0, 0]
        o_ref[...] = jnp.maximum(o, 0.0)

    out = pl.pallas_call(
        body,
        grid=(B // BLK,),
        in_specs=[
            pl.BlockSpec((BLK, EMB), lambda i: (i, 0)),
            pl.BlockSpec((BLK, EMB), lambda i: (i, 0)),
            pl.BlockSpec((EMB, HID), lambda i: (0, 0)),
            pl.BlockSpec((EMB, HID), lambda i: (0, 0)),
            pl.BlockSpec((1, HID), lambda i: (0, 0)),
            pl.BlockSpec((1, HID), lambda i: (0, 0)),
            pl.BlockSpec((1, 1), lambda i: (0, 0)),
        ],
        out_specs=pl.BlockSpec((BLK, 1), lambda i: (i, 0)),
        out_shape=jax.ShapeDtypeStruct((B, 1), jnp.float32),
    )(u_rows, m_rows, w1u_t, w1m_t, b1_2d, w2_2d, b2_2d)
    return out[:, 0]


def kernel(u_idx, m_idx, user_table, movie_table, W1, b1, W2, b2):
    u_idx2d = u_idx.astype(jnp.int32).reshape(B // CHUNK, CHUNK)
    m_idx2d = m_idx.astype(jnp.int32).reshape(B // CHUNK, CHUNK)
    u_rows, m_rows = _sc_gather(u_idx2d, m_idx2d, user_table, movie_table)
    w1u_t = W1[:, :EMB].T
    w1m_t = W1[:, EMB:].T
    return _tc_mlp(u_rows, m_rows, w1u_t, w1m_t,
                   b1.reshape(1, HID), W2, b2.reshape(1, 1))
